# Initial kernel scaffold; baseline (speedup 1.0000x reference)
import jax
import jax.numpy as jnp
from jax.experimental import pallas as pl


def kernel(edges, edge_attr, hidden_features, We1, be1, We2, be2, bn_gamma, bn_beta, Wn1, bn1, Wn2, bn2, Wn3, bn3):
    raise NotImplementedError("write your pallas kernel here")


# speedup vs baseline: 2.3712x; 2.3712x over previous
import functools

import jax
import jax.numpy as jnp
from jax import lax
from jax.experimental import pallas as pl
from jax.experimental.pallas import tpu as pltpu
from jax.experimental.pallas import tpu_sc as plsc

_N = 10000
_E = 320000
_DE = 16
_DH = 128
_MSG = 64

_NC = 2
_NS = 16
_NW = _NC * _NS
_EPW = _E // _NW
_C = 80
_NCH = _EPW // _C
_NPS = _N // _NS

_mesh = plsc.VectorSubcoreMesh(core_axis_name="c", subcore_axis_name="s")


def _proj_body(h_ref, wr_ref, wc_ref, pr_ref, pc_ref):
    h = h_ref[...]
    pr_ref[...] = jnp.dot(h, wr_ref[...], preferred_element_type=jnp.float32)
    pc_ref[...] = jnp.dot(h, wc_ref[...], preferred_element_type=jnp.float32)


_proj_call = pl.pallas_call(
    _proj_body,
    out_shape=[
        jax.ShapeDtypeStruct((_N, _MSG), jnp.float32),
        jax.ShapeDtypeStruct((_N, _MSG), jnp.float32),
    ],
)


@functools.partial(
    pl.kernel,
    mesh=_mesh,
    compiler_params=pltpu.CompilerParams(use_tc_tiling_on_sc=False),
    out_type=[
        jax.ShapeDtypeStruct((_E, _MSG), jnp.float32),
        jax.ShapeDtypeStruct((_E, _MSG), jnp.float32),
    ],
    scratch_types=[
        pltpu.VMEM((_EPW,), jnp.int32),
        pltpu.VMEM((_EPW,), jnp.int32),
        pltpu.VMEM((_C, _MSG), jnp.float32),
        pltpu.VMEM((_C, _MSG), jnp.float32),
        pltpu.SemaphoreType.DMA,
        pltpu.SemaphoreType.DMA,
    ],
)
def _sc_gather(row_hbm, col_hbm, pr_hbm, pc_hbm, gr_hbm, gc_hbm,
               idxr_v, idxc_v, bufr_v, bufc_v, semr, semc):
    wid = lax.axis_index("s") * _NC + lax.axis_index("c")
    base = wid * _EPW
    pltpu.sync_copy(row_hbm.at[pl.ds(base, _EPW)], idxr_v)
    pltpu.sync_copy(col_hbm.at[pl.ds(base, _EPW)], idxc_v)

    def body(j, carry):
        off = j * _C
        cr = pltpu.async_copy(pr_hbm.at[idxr_v.at[pl.ds(off, _C)]], bufr_v, semr)
        cc = pltpu.async_copy(pc_hbm.at[idxc_v.at[pl.ds(off, _C)]], bufc_v, semc)
        cr.wait()
        cc.wait()
        pltpu.sync_copy(bufr_v, gr_hbm.at[pl.ds(base + off, _C)])
        pltpu.sync_copy(bufc_v, gc_hbm.at[pl.ds(base + off, _C)])
        return carry

    lax.fori_loop(0, _NCH, body, 0)


_BE = 4000


def _edge_body(gr_ref, gc_ref, ea_ref, we_ref, be1_ref, we2_ref, be2_ref,
               m_ref):
    pre = (gr_ref[...] + gc_ref[...]
           + jnp.dot(ea_ref[...], we_ref[...],
                     preferred_element_type=jnp.float32)
           + be1_ref[...])
    t = pre * jax.nn.sigmoid(pre)
    u = jnp.dot(t, we2_ref[...], preferred_element_type=jnp.float32) \
        + be2_ref[...]
    m_ref[...] = u * jax.nn.sigmoid(u)


_edge_call = pl.pallas_call(
    _edge_body,
    grid=(_E // _BE,),
    in_specs=[
        pl.BlockSpec((_BE, _MSG), lambda i: (i, 0)),
        pl.BlockSpec((_BE, _MSG), lambda i: (i, 0)),
        pl.BlockSpec((_BE, _DE), lambda i: (i, 0)),
        pl.BlockSpec((_DE, _MSG), lambda i: (0, 0)),
        pl.BlockSpec((1, _MSG), lambda i: (0, 0)),
        pl.BlockSpec((_MSG, _MSG), lambda i: (0, 0)),
        pl.BlockSpec((1, _MSG), lambda i: (0, 0)),
    ],
    out_specs=pl.BlockSpec((_BE, _MSG), lambda i: (i, 0)),
    out_shape=jax.ShapeDtypeStruct((_E, _MSG), jnp.float32),
)


@functools.partial(
    pl.kernel,
    mesh=_mesh,
    compiler_params=pltpu.CompilerParams(use_tc_tiling_on_sc=False),
    out_type=jax.ShapeDtypeStruct((_NC * _N, _MSG), jnp.float32),
    scratch_types=[
        pltpu.VMEM((_C,), jnp.int32),
        pltpu.VMEM((_C, _MSG), jnp.float32),
        pltpu.VMEM((_NPS, _MSG), jnp.float32),
        pltpu.VMEM_SHARED((_N, _MSG), jnp.float32),
    ],
)
def _sc_scatter(row_hbm, m_hbm, out_hbm, idx_v, m_v, stage_v, acc_sh):
    c = lax.axis_index("c")
    s = lax.axis_index("s")

    zeros = jnp.zeros((16,), jnp.float32)

    def zbody(i, carry):
        for k in range(_MSG // 16):
            stage_v[i, pl.ds(k * 16, 16)] = zeros
        return carry

    lax.fori_loop(0, _NPS, zbody, 0)
    pltpu.sync_copy(stage_v, acc_sh.at[pl.ds(s * _NPS, _NPS)])
    plsc.subcore_barrier()

    base = c * (_E // _NC) + s * _EPW

    def body(j, carry):
        off = base + j * _C
        pltpu.sync_copy(row_hbm.at[pl.ds(off, _C)], idx_v)
        pltpu.sync_copy(m_hbm.at[pl.ds(off, _C)], m_v)
        pltpu.sync_copy(m_v, acc_sh.at[idx_v], add=True)
        return carry

    lax.fori_loop(0, _NCH, body, 0)
    plsc.subcore_barrier()
    pltpu.sync_copy(acc_sh.at[pl.ds(s * _NPS, _NPS)],
                    out_hbm.at[pl.ds(c * _N + s * _NPS, _NPS)])


def _node_body(p_ref, h_ref, g_ref, b_ref, w1_ref, b1_ref, w2_ref, b2_ref,
               w3_ref, b3_ref, o_ref):
    summed = p_ref[0:_N, :] + p_ref[_N:2 * _N, :]
    h = h_ref[...]
    x = jnp.concatenate([summed, h], axis=1)
    mean = jnp.mean(x, axis=0, keepdims=True)
    xc = x - mean
    var = jnp.mean(xc * xc, axis=0, keepdims=True)
    xn = xc * (g_ref[...] * jax.lax.rsqrt(var + 1e-5)) + b_ref[...]
    t = jnp.dot(xn, w1_ref[...], preferred_element_type=jnp.float32) \
        + b1_ref[...]
    t = t * jax.nn.sigmoid(t)
    t = jnp.dot(t, w2_ref[...], preferred_element_type=jnp.float32) \
        + b2_ref[...]
    t = t * jax.nn.sigmoid(t)
    y = jnp.dot(t, w3_ref[...], preferred_element_type=jnp.float32) \
        + b3_ref[...]
    o_ref[...] = h + y


_node_call = pl.pallas_call(
    _node_body,
    out_shape=jax.ShapeDtypeStruct((_N, _DH), jnp.float32),
)


def kernel(edges, edge_attr, hidden_features, We1, be1, We2, be2,
           bn_gamma, bn_beta, Wn1, bn1, Wn2, bn2, Wn3, bn3):
    row = edges[0].astype(jnp.int32)
    col = edges[1].astype(jnp.int32)
    wr = We1[_DE:_DE + _DH]
    wc = We1[_DE + _DH:]
    we = We1[:_DE]

    pr, pc = _proj_call(hidden_features, wr, wc)
    gr, gc = _sc_gather(row, col, pr, pc)
    m = _edge_call(gr, gc, edge_attr,
                   we, be1.reshape(1, _MSG), We2, be2.reshape(1, _MSG))
    partials = _sc_scatter(row, m)
    out = _node_call(partials, hidden_features,
                     bn_gamma.reshape(1, _MSG + _DH),
                     bn_beta.reshape(1, _MSG + _DH),
                     Wn1, bn1.reshape(1, -1), Wn2, bn2.reshape(1, -1),
                     Wn3, bn3.reshape(1, -1))
    return out


# speedup vs baseline: 4.1520x; 1.7511x over previous
import functools

import jax
import jax.numpy as jnp
from jax import lax
from jax.experimental import pallas as pl
from jax.experimental.pallas import tpu as pltpu
from jax.experimental.pallas import tpu_sc as plsc

_N = 10000
_E = 320000
_DE = 16
_DH = 128
_MSG = 64

_NC = 2
_NS = 16
_NW = _NC * _NS
_EPW = _E // _NW
_C = 80
_NCH = _EPW // _C
_NPS = _N // _NS

_mesh = plsc.VectorSubcoreMesh(core_axis_name="c", subcore_axis_name="s")


def _proj_body(h_ref, wr_ref, wc_ref, pr_ref, pc_ref):
    h = h_ref[...]
    pr_ref[...] = jnp.dot(h, wr_ref[...], preferred_element_type=jnp.float32)
    pc_ref[...] = jnp.dot(h, wc_ref[...], preferred_element_type=jnp.float32)


_proj_call = pl.pallas_call(
    _proj_body,
    out_shape=[
        jax.ShapeDtypeStruct((_N, _MSG), jnp.float32),
        jax.ShapeDtypeStruct((_N, _MSG), jnp.float32),
    ],
)


_K = 5
_PHE = _K * _C
_PH = _EPW // _PHE


@functools.partial(
    pl.kernel,
    mesh=_mesh,
    compiler_params=pltpu.CompilerParams(use_tc_tiling_on_sc=False),
    out_type=jax.ShapeDtypeStruct((_E, 2 * _MSG), jnp.float32),
    scratch_types=[
        pltpu.VMEM((_EPW,), jnp.int32),
        pltpu.VMEM((_EPW,), jnp.int32),
        pltpu.VMEM((_PHE, _MSG), jnp.float32),
        pltpu.VMEM((_PHE, _MSG), jnp.float32),
        pltpu.VMEM((_PHE, _MSG), jnp.float32),
        pltpu.VMEM((_PHE, _MSG), jnp.float32),
        pltpu.SemaphoreType.DMA,
        pltpu.SemaphoreType.DMA,
        pltpu.SemaphoreType.DMA,
        pltpu.SemaphoreType.DMA,
    ],
)
def _sc_gather(row_hbm, col_hbm, pr_hbm, pc_hbm, g_hbm,
               idxr_v, idxc_v, bufr0, bufc0, bufr1, bufc1,
               gsem0, gsem1, wsem0, wsem1):
    wid = lax.axis_index("s") * _NC + lax.axis_index("c")
    base = wid * _EPW
    pltpu.sync_copy(row_hbm.at[pl.ds(base, _EPW)], idxr_v)
    pltpu.sync_copy(col_hbm.at[pl.ds(base, _EPW)], idxc_v)

    def fire_gathers(p, br, bc, gsem):
        for k in range(_K):
            off = p * _PHE + k * _C
            pltpu.async_copy(pr_hbm.at[idxr_v.at[pl.ds(off, _C)]],
                             br.at[pl.ds(k * _C, _C)], gsem)
            pltpu.async_copy(pc_hbm.at[idxc_v.at[pl.ds(off, _C)]],
                             bc.at[pl.ds(k * _C, _C)], gsem)

    def drain_gathers(p, br, bc, gsem):
        for k in range(_K):
            off = p * _PHE + k * _C
            pltpu.make_async_copy(pr_hbm.at[idxr_v.at[pl.ds(off, _C)]],
                                  br.at[pl.ds(k * _C, _C)], gsem).wait()
            pltpu.make_async_copy(pc_hbm.at[idxc_v.at[pl.ds(off, _C)]],
                                  bc.at[pl.ds(k * _C, _C)], gsem).wait()

    def fire_writes(p, br, bc, wsem):
        off = base + p * _PHE
        pltpu.async_copy(
            br, g_hbm.at[pl.ds(off, _PHE), pl.ds(0, _MSG)], wsem)
        pltpu.async_copy(
            bc, g_hbm.at[pl.ds(off, _PHE), pl.ds(_MSG, _MSG)], wsem)

    def drain_writes(p, br, bc, wsem):
        off = base + p * _PHE
        pltpu.make_async_copy(
            br, g_hbm.at[pl.ds(off, _PHE), pl.ds(0, _MSG)], wsem).wait()
        pltpu.make_async_copy(
            bc, g_hbm.at[pl.ds(off, _PHE), pl.ds(_MSG, _MSG)], wsem).wait()

    fire_gathers(0, bufr0, bufc0, gsem0)

    def phase(p, carry):
        def step(br, bc, gsem, wsem, obr, obc, ogsem, owsem):
            drain_gathers(p, br, bc, gsem)
            fire_writes(p, br, bc, wsem)

            @pl.when(p >= 1)
            def _():
                drain_writes(p - 1, obr, obc, owsem)

            @pl.when(p + 1 < _PH)
            def _():
                fire_gathers(p + 1, obr, obc, ogsem)

        parity = lax.rem(p, 2)

        @pl.when(parity == 0)
        def _():
            step(bufr0, bufc0, gsem0, wsem0, bufr1, bufc1, gsem1, wsem1)

        @pl.when(parity == 1)
        def _():
            step(bufr1, bufc1, gsem1, wsem1, bufr0, bufc0, gsem0, wsem0)

        return carry

    lax.fori_loop(0, _PH, phase, 0)
    drain_writes(_PH - 1, bufr0, bufc0, wsem0)


_BE = 4000


def _edge_body(g_ref, ea_ref, we_ref, be1_ref, we2_ref, be2_ref,
               m_ref):
    pre = (g_ref[:, :_MSG] + g_ref[:, _MSG:]
           + jnp.dot(ea_ref[...], we_ref[...],
                     preferred_element_type=jnp.float32)
           + be1_ref[...])
    t = pre * jax.nn.sigmoid(pre)
    u = jnp.dot(t, we2_ref[...], preferred_element_type=jnp.float32) \
        + be2_ref[...]
    m_ref[...] = u * jax.nn.sigmoid(u)


_edge_call = pl.pallas_call(
    _edge_body,
    grid=(_E // _BE,),
    in_specs=[
        pl.BlockSpec((_BE, 2 * _MSG), lambda i: (i, 0)),
        pl.BlockSpec((_BE, _DE), lambda i: (i, 0)),
        pl.BlockSpec((_DE, _MSG), lambda i: (0, 0)),
        pl.BlockSpec((1, _MSG), lambda i: (0, 0)),
        pl.BlockSpec((_MSG, _MSG), lambda i: (0, 0)),
        pl.BlockSpec((1, _MSG), lambda i: (0, 0)),
    ],
    out_specs=pl.BlockSpec((_BE, _MSG), lambda i: (i, 0)),
    out_shape=jax.ShapeDtypeStruct((_E, _MSG), jnp.float32),
)


@functools.partial(
    pl.kernel,
    mesh=_mesh,
    compiler_params=pltpu.CompilerParams(use_tc_tiling_on_sc=False),
    out_type=jax.ShapeDtypeStruct((_NC * _N, _MSG), jnp.float32),
    scratch_types=[
        pltpu.VMEM((_K, _C), jnp.int32),
        pltpu.VMEM((_K, _C), jnp.int32),
        pltpu.VMEM((_PHE, _MSG), jnp.float32),
        pltpu.VMEM((_PHE, _MSG), jnp.float32),
        pltpu.VMEM_SHARED((_N, _MSG), jnp.float32),
        pltpu.SemaphoreType.DMA,
        pltpu.SemaphoreType.DMA,
    ],
)
def _sc_scatter(row_hbm, m_hbm, out_hbm, idx0, idx1, mb0, mb1,
                acc_sh, lsem0, lsem1):
    c = lax.axis_index("c")
    s = lax.axis_index("s")

    zeros = jnp.zeros((16,), jnp.float32)

    def zbody(i, carry):
        for k in range(_MSG // 16):
            mb0[i, pl.ds(k * 16, 16)] = zeros
        return carry

    lax.fori_loop(0, _PHE, zbody, 0)
    pltpu.sync_copy(mb0, acc_sh.at[pl.ds(s * _NPS, _PHE)])
    pltpu.sync_copy(mb0.at[pl.ds(0, _NPS - _PHE)],
                    acc_sh.at[pl.ds(s * _NPS + _PHE, _NPS - _PHE)])
    plsc.subcore_barrier()

    base = c * (_E // _NC) + s * _EPW

    def fire_loads(p, idxb, mb, lsem):
        off = base + p * _PHE
        pltpu.async_copy(m_hbm.at[pl.ds(off, _PHE)], mb, lsem)
        for k in range(_K):
            pltpu.async_copy(row_hbm.at[pl.ds(off + k * _C, _C)],
                             idxb.at[k], lsem)

    def drain_loads(p, idxb, mb, lsem):
        off = base + p * _PHE
        pltpu.make_async_copy(m_hbm.at[pl.ds(off, _PHE)], mb, lsem).wait()
        for k in range(_K):
            pltpu.make_async_copy(row_hbm.at[pl.ds(off + k * _C, _C)],
                                  idxb.at[k], lsem).wait()

    def scatter_adds(idxb, mb):
        for k in range(_K):
            pltpu.sync_copy(mb.at[pl.ds(k * _C, _C)], acc_sh.at[idxb.at[k]],
                            add=True)

    fire_loads(0, idx0, mb0, lsem0)

    def phase(p, carry):
        def step(idxb, mb, lsem, oidxb, omb, olsem):
            drain_loads(p, idxb, mb, lsem)

            @pl.when(p + 1 < _PH)
            def _():
                fire_loads(p + 1, oidxb, omb, olsem)

            scatter_adds(idxb, mb)

        parity = lax.rem(p, 2)

        @pl.when(parity == 0)
        def _():
            step(idx0, mb0, lsem0, idx1, mb1, lsem1)

        @pl.when(parity == 1)
        def _():
            step(idx1, mb1, lsem1, idx0, mb0, lsem0)

        return carry

    lax.fori_loop(0, _PH, phase, 0)
    plsc.subcore_barrier()
    pltpu.sync_copy(acc_sh.at[pl.ds(s * _NPS, _NPS)],
                    out_hbm.at[pl.ds(c * _N + s * _NPS, _NPS)])


def _node_body(p_ref, h_ref, g_ref, b_ref, w1_ref, b1_ref, w2_ref, b2_ref,
               w3_ref, b3_ref, o_ref):
    summed = p_ref[0:_N, :] + p_ref[_N:2 * _N, :]
    h = h_ref[...]
    x = jnp.concatenate([summed, h], axis=1)
    mean = jnp.mean(x, axis=0, keepdims=True)
    xc = x - mean
    var = jnp.mean(xc * xc, axis=0, keepdims=True)
    xn = xc * (g_ref[...] * jax.lax.rsqrt(var + 1e-5)) + b_ref[...]
    t = jnp.dot(xn, w1_ref[...], preferred_element_type=jnp.float32) \
        + b1_ref[...]
    t = t * jax.nn.sigmoid(t)
    t = jnp.dot(t, w2_ref[...], preferred_element_type=jnp.float32) \
        + b2_ref[...]
    t = t * jax.nn.sigmoid(t)
    y = jnp.dot(t, w3_ref[...], preferred_element_type=jnp.float32) \
        + b3_ref[...]
    o_ref[...] = h + y


_node_call = pl.pallas_call(
    _node_body,
    out_shape=jax.ShapeDtypeStruct((_N, _DH), jnp.float32),
)


def kernel(edges, edge_attr, hidden_features, We1, be1, We2, be2,
           bn_gamma, bn_beta, Wn1, bn1, Wn2, bn2, Wn3, bn3):
    row = edges[0].astype(jnp.int32)
    col = edges[1].astype(jnp.int32)
    wr = We1[_DE:_DE + _DH]
    wc = We1[_DE + _DH:]
    we = We1[:_DE]

    pr, pc = _proj_call(hidden_features, wr, wc)
    g = _sc_gather(row, col, pr, pc)
    m = _edge_call(g, edge_attr,
                   we, be1.reshape(1, _MSG), We2, be2.reshape(1, _MSG))
    partials = _sc_scatter(row, m)
    out = _node_call(partials, hidden_features,
                     bn_gamma.reshape(1, _MSG + _DH),
                     bn_beta.reshape(1, _MSG + _DH),
                     Wn1, bn1.reshape(1, -1), Wn2, bn2.reshape(1, -1),
                     Wn3, bn3.reshape(1, -1))
    return out
